# baseline (speedup 1.0000x reference)
import math

import jax
import jax.numpy as jnp
from jax import lax
from jax.experimental import pallas as pl
from jax.experimental.pallas import tpu as pltpu
from jax.experimental.pallas import tpu_sc as plsc

N, E, D = 10000, 320000, 128
BETA = 1.0 / 3.0
GAMMA = -0.2
ZETA = 1.2
LOC_BIAS = 2.0
PEN_SHIFT = LOC_BIAS - BETA * math.log(-GAMMA / ZETA)

NC, NS, L = 2, 16, 16
CH = 128
NCHUNK = 160
EPT = NCHUNK * CH
EPAD = NS * EPT
HALF = D // 2
NP = 10240
RPT = NP // NS
NBUF = 2
A = 1


def _sigmoid(v):
    return 1.0 / (1.0 + jnp.exp(-v))


def _body(x2, srcp, dstp, lgp, fm, out2, pen,
          src_v, lg_v, dst_v, fm_v, rows0, rows1, pen_v, acc,
          gs0, gs1, ss0, ss1):
    c = lax.axis_index("c")
    s = lax.axis_index("s")
    ebase = s * EPT
    rows = (rows0, rows1)
    gsems = (gs0, gs1)
    ssems = (ss0, ss1)

    pltpu.sync_copy(srcp.at[pl.ds(ebase, EPT)], src_v)
    pltpu.sync_copy(lgp.at[pl.ds(ebase, EPT)], lg_v)
    pltpu.sync_copy(dstp.at[pl.ds(s * NCHUNK, NCHUNK)], dst_v)
    pltpu.sync_copy(fm, fm_v)

    zero16 = jnp.zeros((L,), jnp.float32)

    def zrow(i, carry):
        for q in range(HALF // L):
            rows0[i, pl.ds(q * L, L)] = zero16
        return carry

    lax.fori_loop(0, CH, zrow, 0)
    r0 = s * RPT
    for k in range(RPT // CH):
        pltpu.sync_copy(rows0, acc.at[pl.ds(r0 + k * CH, CH)])

    def sfm(i, carry):
        sl = pl.ds(i * L, L)
        fm_v[sl] = _sigmoid(fm_v[sl])
        return carry

    lax.fori_loop(0, N // L, sfm, 0)

    def edge16(i, pacc):
        sl = pl.ds(i * L, L)
        sv = src_v[sl]
        lgv = lg_v[sl]
        fmg = plsc.load_gather(fm_v, [sv])
        gate = jnp.clip(_sigmoid(lgv + LOC_BIAS) * (ZETA - GAMMA) + GAMMA,
                        0.0, 1.0)
        lg_v[sl] = gate * fmg
        src_v[sl] = sv * 2 + c
        return pacc + _sigmoid(lgv + PEN_SHIFT)

    pen16 = lax.fori_loop(0, EPT // L, edge16, jnp.zeros((L,), jnp.float32))
    pen_v[...] = pen16
    pltpu.sync_copy(pen_v, pen.at[pl.ds((c * NS + s) * L, L)])

    plsc.subcore_barrier()

    def issue_gather(j, b):
        pltpu.async_copy(x2.at[src_v.at[pl.ds(j * CH, CH)]], rows[b],
                         gsems[b])

    def wait_chunk(b):
        pltpu.make_async_copy(x2.at[pl.ds(0, CH)], rows[b],
                              gsems[b]).wait()

    def wait_rows_dma(b, sem):
        pltpu.make_async_copy(x2.at[pl.ds(0, CH)], rows[b], sem).wait()

    for b in range(A):
        issue_gather(b, b)

    lane_idx = [jnp.full((L,), k, jnp.int32) for k in range(L)]

    def step(t, carry):
        for b in range(NBUF):
            j = t * NBUF + b
            wait_chunk(b)

            def grp(g, icarry):
                wv = lg_v[pl.ds(j * CH + g * L, L)]
                for k in range(L):
                    e = g * L + k
                    w16 = wv.at[lane_idx[k]].get(mode='promise_in_bounds')
                    for q in range(HALF // L):
                        sl = pl.ds(q * L, L)
                        rows[b][e, sl] = rows[b][e, sl] * w16
                return icarry

            lax.fori_loop(0, CH // L, grp, 0, unroll=True)
            pltpu.async_copy(rows[b], acc.at[dst_v.at[j]], ssems[b],
                             add=True)

            bp = (b + A) % NBUF
            if b < A:
                @pl.when(t >= 1)
                def _drain():
                    wait_rows_dma(bp, ssems[bp])

                issue_gather(j + A, bp)
            else:
                wait_rows_dma(bp, ssems[bp])

                @pl.when(j + A <= NCHUNK - 1)
                def _refill():
                    issue_gather(j + A, bp)

        return carry

    lax.fori_loop(0, NCHUNK // NBUF, step, 0)

    for j in range(NCHUNK - A, NCHUNK):
        b = j % NBUF
        wait_rows_dma(b, ssems[b])

    plsc.subcore_barrier()

    pltpu.sync_copy(acc.at[pl.ds(r0, RPT)],
                    out2.at[pl.ds(c * NP + r0, RPT)])


_sc_call = pl.kernel(
    _body,
    out_type=(
        jax.ShapeDtypeStruct((NC * NP, HALF), jnp.float32),
        jax.ShapeDtypeStruct((NC * NS * L,), jnp.float32),
    ),
    mesh=plsc.VectorSubcoreMesh(core_axis_name="c", subcore_axis_name="s"),
    compiler_params=pltpu.CompilerParams(
        needs_layout_passes=False, use_tc_tiling_on_sc=False),
    scratch_types=[
        pltpu.VMEM((EPT,), jnp.int32),
        pltpu.VMEM((EPT,), jnp.float32),
        pltpu.VMEM((NCHUNK, CH), jnp.int32),
        pltpu.VMEM((N,), jnp.float32),
        pltpu.VMEM((CH, HALF), jnp.float32),
        pltpu.VMEM((CH, HALF), jnp.float32),
        pltpu.VMEM((L,), jnp.float32),
        pltpu.VMEM_SHARED((NP, HALF), jnp.float32),
        pltpu.SemaphoreType.DMA,
        pltpu.SemaphoreType.DMA,
        pltpu.SemaphoreType.DMA,
        pltpu.SemaphoreType.DMA,
    ],
)


def kernel(x, edge_index, gate_logits, feat_mask):
    x2 = x.reshape(NC * N, HALF)
    pad = EPAD - E
    src = jnp.concatenate([edge_index[0], jnp.zeros((pad,), jnp.int32)])
    dst = jnp.concatenate([edge_index[1], jnp.zeros((pad,), jnp.int32)])
    lg = jnp.concatenate(
        [gate_logits, jnp.full((pad,), -1e30, jnp.float32)])
    dst2d = dst.reshape(EPAD // CH, CH)

    out2, pen = _sc_call(x2, src, dst2d, lg, feat_mask)
    out = jnp.concatenate([out2[:N], out2[NP:NP + N]], axis=1)
    penalty = jnp.sum(pen) / (NC * E)
    return out, penalty


# speedup vs baseline: 1.0908x; 1.0908x over previous
import math

import jax
import jax.numpy as jnp
from jax import lax
from jax.experimental import pallas as pl
from jax.experimental.pallas import tpu as pltpu
from jax.experimental.pallas import tpu_sc as plsc

N, E, D = 10000, 320000, 128
BETA = 1.0 / 3.0
GAMMA = -0.2
ZETA = 1.2
LOC_BIAS = 2.0
PEN_SHIFT = LOC_BIAS - BETA * math.log(-GAMMA / ZETA)

NC, NS, L = 2, 16, 16
CH = 128
NCHUNK = 160
EPT = NCHUNK * CH
EPAD = NS * EPT
HALF = D // 2
NP = 10240
RPT = NP // NS
NBUF = 4
A = 2


def _sigmoid(v):
    return 1.0 / (1.0 + jnp.exp(-v))


def _body(x2, srcp, dstp, lgp, fm, out2, pen,
          src_v, lg_v, dstb, fm_v, rows0, rows1, rows2, rows3, pen_v, acc,
          gs0, gs1, gs2, gs3, ss0, ss1, ss2, ss3):
    c = lax.axis_index("c")
    s = lax.axis_index("s")
    ebase = s * EPT
    rows = (rows0, rows1, rows2, rows3)
    gsems = (gs0, gs1, gs2, gs3)
    ssems = (ss0, ss1, ss2, ss3)

    pltpu.sync_copy(srcp.at[pl.ds(ebase, EPT)], src_v)
    pltpu.sync_copy(lgp.at[pl.ds(ebase, EPT)], lg_v)
    pltpu.sync_copy(fm, fm_v)

    zero16 = jnp.zeros((L,), jnp.float32)

    def zrow(i, carry):
        for q in range(HALF // L):
            rows0[i, pl.ds(q * L, L)] = zero16
        return carry

    lax.fori_loop(0, CH, zrow, 0)
    r0 = s * RPT
    for k in range(RPT // CH):
        pltpu.sync_copy(rows0, acc.at[pl.ds(r0 + k * CH, CH)])

    def sfm(i, carry):
        sl = pl.ds(i * L, L)
        fm_v[sl] = _sigmoid(fm_v[sl])
        return carry

    lax.fori_loop(0, N // L, sfm, 0)

    def edge16(i, pacc):
        sl = pl.ds(i * L, L)
        sv = src_v[sl]
        lgv = lg_v[sl]
        fmg = plsc.load_gather(fm_v, [sv])
        gate = jnp.clip(_sigmoid(lgv + LOC_BIAS) * (ZETA - GAMMA) + GAMMA,
                        0.0, 1.0)
        lg_v[sl] = gate * fmg
        src_v[sl] = sv * 2 + c
        return pacc + _sigmoid(lgv + PEN_SHIFT)

    pen16 = lax.fori_loop(0, EPT // L, edge16, jnp.zeros((L,), jnp.float32))
    pen_v[...] = pen16
    pltpu.sync_copy(pen_v, pen.at[pl.ds((c * NS + s) * L, L)])

    plsc.subcore_barrier()

    def issue_gather(j, b):
        pltpu.async_copy(x2.at[src_v.at[pl.ds(j * CH, CH)]], rows[b],
                         gsems[b])
        pltpu.async_copy(dstp.at[pl.ds(s * NCHUNK + j, 1)], dstb.at[b],
                         gsems[b])

    def wait_chunk(b):
        pltpu.make_async_copy(x2.at[pl.ds(0, CH)], rows[b],
                              gsems[b]).wait()
        pltpu.make_async_copy(dstp.at[pl.ds(0, 1)], dstb.at[b],
                              gsems[b]).wait()

    def wait_rows_dma(b, sem):
        pltpu.make_async_copy(x2.at[pl.ds(0, CH)], rows[b], sem).wait()

    for b in range(A):
        issue_gather(b, b)

    lane_idx = [jnp.full((L,), k, jnp.int32) for k in range(L)]

    def step(t, carry):
        for b in range(NBUF):
            j = t * NBUF + b
            wait_chunk(b)

            def grp(g, icarry):
                wv = lg_v[pl.ds(j * CH + g * L, L)]
                for k in range(L):
                    e = g * L + k
                    w16 = wv.at[lane_idx[k]].get(mode='promise_in_bounds')
                    for q in range(HALF // L):
                        sl = pl.ds(q * L, L)
                        rows[b][e, sl] = rows[b][e, sl] * w16
                return icarry

            lax.fori_loop(0, CH // L, grp, 0, unroll=True)
            pltpu.async_copy(rows[b], acc.at[dstb.at[b, 0]], ssems[b],
                             add=True)

            bp = (b + A) % NBUF
            if b < A:
                @pl.when(t >= 1)
                def _drain():
                    wait_rows_dma(bp, ssems[bp])

                issue_gather(j + A, bp)
            else:
                wait_rows_dma(bp, ssems[bp])

                @pl.when(j + A <= NCHUNK - 1)
                def _refill():
                    issue_gather(j + A, bp)

        return carry

    lax.fori_loop(0, NCHUNK // NBUF, step, 0)

    for j in range(NCHUNK - A, NCHUNK):
        b = j % NBUF
        wait_rows_dma(b, ssems[b])

    plsc.subcore_barrier()

    pltpu.sync_copy(acc.at[pl.ds(r0, RPT)],
                    out2.at[pl.ds(c * NP + r0, RPT)])


_sc_call = pl.kernel(
    _body,
    out_type=(
        jax.ShapeDtypeStruct((NC * NP, HALF), jnp.float32),
        jax.ShapeDtypeStruct((NC * NS * L,), jnp.float32),
    ),
    mesh=plsc.VectorSubcoreMesh(core_axis_name="c", subcore_axis_name="s"),
    compiler_params=pltpu.CompilerParams(
        needs_layout_passes=False, use_tc_tiling_on_sc=False),
    scratch_types=[
        pltpu.VMEM((EPT,), jnp.int32),
        pltpu.VMEM((EPT,), jnp.float32),
        pltpu.VMEM((NBUF, 1, CH), jnp.int32),
        pltpu.VMEM((N,), jnp.float32),
        pltpu.VMEM((CH, HALF), jnp.float32),
        pltpu.VMEM((CH, HALF), jnp.float32),
        pltpu.VMEM((CH, HALF), jnp.float32),
        pltpu.VMEM((CH, HALF), jnp.float32),
        pltpu.VMEM((L,), jnp.float32),
        pltpu.VMEM_SHARED((NP, HALF), jnp.float32),
        pltpu.SemaphoreType.DMA,
        pltpu.SemaphoreType.DMA,
        pltpu.SemaphoreType.DMA,
        pltpu.SemaphoreType.DMA,
        pltpu.SemaphoreType.DMA,
        pltpu.SemaphoreType.DMA,
        pltpu.SemaphoreType.DMA,
        pltpu.SemaphoreType.DMA,
    ],
)


def kernel(x, edge_index, gate_logits, feat_mask):
    x2 = x.reshape(NC * N, HALF)
    pad = EPAD - E
    src = jnp.concatenate([edge_index[0], jnp.zeros((pad,), jnp.int32)])
    dst = jnp.concatenate([edge_index[1], jnp.zeros((pad,), jnp.int32)])
    lg = jnp.concatenate(
        [gate_logits, jnp.full((pad,), -1e30, jnp.float32)])
    dst2d = dst.reshape(EPAD // CH, CH)

    out2, pen = _sc_call(x2, src, dst2d, lg, feat_mask)
    out = jnp.concatenate([out2[:N], out2[NP:NP + N]], axis=1)
    penalty = jnp.sum(pen) / (NC * E)
    return out, penalty


# speedup vs baseline: 2.0825x; 1.9091x over previous
import math

import jax
import jax.numpy as jnp
from jax import lax
from jax.experimental import pallas as pl
from jax.experimental.pallas import tpu as pltpu
from jax.experimental.pallas import tpu_sc as plsc

N, E, D = 10000, 320000, 128
BETA = 1.0 / 3.0
GAMMA = -0.2
ZETA = 1.2
LOC_BIAS = 2.0
PEN_SHIFT = LOC_BIAS - BETA * math.log(-GAMMA / ZETA)

NC, NS, L = 2, 16, 16
CH = 128
NCHUNK = 160
EPT = NCHUNK * CH
EPAD = NS * EPT
HALF = D // 2
NP = 10240
RPT = NP // NS
NBUF = 3
A = 2
SD = 6
XB = RPT // CH
UN = 6


def _sigmoid(v):
    return 1.0 / (1.0 + jnp.exp(-v))


def _body(x2, srcp, dstp, lgp, fm, out2, pen,
          lg_v, srcb, dstb, fmb, xidx, fidx, rows0, rows1, rows2, pen_v,
          acc, xcache,
          gs0, gs1, gs2, ss0, ss1, ss2, is0, is1, is2, is3, is4, is5):
    c = lax.axis_index("c")
    s = lax.axis_index("s")
    ebase = s * EPT
    rows = (rows0, rows1, rows2)
    gsems = (gs0, gs1, gs2)
    ssems = (ss0, ss1, ss2)
    isems = (is0, is1, is2, is3, is4, is5)

    pltpu.sync_copy(lgp.at[pl.ds(ebase, EPT)], lg_v)

    lane_idx = [jnp.full((L,), k, jnp.int32) for k in range(L)]

    for blk in range(XB):
        u0 = s * RPT + blk * CH

        def bidx(q, carry):
            io = lax.iota(jnp.int32, L)
            u = u0 + q * L + io
            xidx[pl.ds(q * L, L)] = jnp.minimum(u * 2 + c, 2 * N - 1)
            fidx[pl.ds(q * L, L)] = jnp.minimum(u, N - 1)
            return carry

        lax.fori_loop(0, CH // L, bidx, 0, unroll=True)
        pltpu.async_copy(x2.at[xidx], rows0, gs0)
        pltpu.sync_copy(fm.at[fidx], fmb)
        pltpu.make_async_copy(x2.at[pl.ds(0, CH)], rows0, gs0).wait()

        def sfm(q, carry):
            sl = pl.ds(q * L, L)
            fmb[sl] = _sigmoid(fmb[sl])
            return carry

        lax.fori_loop(0, CH // L, sfm, 0)

        def scale(g, carry):
            wv = fmb[pl.ds(g * L, L)]
            for k in range(L):
                e = g * L + k
                w16 = wv.at[lane_idx[k]].get(mode='promise_in_bounds')
                for q in range(HALF // L):
                    sl = pl.ds(q * L, L)
                    rows0[e, sl] = rows0[e, sl] * w16
            return carry

        lax.fori_loop(0, CH // L, scale, 0)
        pltpu.sync_copy(rows0, xcache.at[pl.ds(u0, CH)])

    zero16 = jnp.zeros((L,), jnp.float32)

    def zrow(i, carry):
        for q in range(HALF // L):
            rows0[i, pl.ds(q * L, L)] = zero16
        return carry

    lax.fori_loop(0, CH, zrow, 0)
    r0 = s * RPT
    for k in range(RPT // CH):
        pltpu.sync_copy(rows0, acc.at[pl.ds(r0 + k * CH, CH)])

    def edge16(i, pacc):
        sl = pl.ds(i * L, L)
        lgv = lg_v[sl]
        gate = jnp.clip(_sigmoid(lgv + LOC_BIAS) * (ZETA - GAMMA) + GAMMA,
                        0.0, 1.0)
        lg_v[sl] = gate
        return pacc + _sigmoid(lgv + PEN_SHIFT)

    pen16 = lax.fori_loop(0, EPT // L, edge16, jnp.zeros((L,), jnp.float32))
    pen_v[...] = pen16
    pltpu.sync_copy(pen_v, pen.at[pl.ds((c * NS + s) * L, L)])

    plsc.subcore_barrier()

    def issue_idx(j, m):
        pltpu.async_copy(srcp.at[pl.ds(s * NCHUNK + j, 1)], srcb.at[m],
                         isems[m])
        pltpu.async_copy(dstp.at[pl.ds(s * NCHUNK + j, 1)], dstb.at[m],
                         isems[m])

    def wait_idx(m):
        pltpu.make_async_copy(srcp.at[pl.ds(0, 1)], srcb.at[m],
                              isems[m]).wait()
        pltpu.make_async_copy(dstp.at[pl.ds(0, 1)], dstb.at[m],
                              isems[m]).wait()

    def issue_gather(m, b):
        pltpu.async_copy(xcache.at[srcb.at[m, 0]], rows[b], gsems[b])

    def wait_rows_dma(b, sem):
        pltpu.make_async_copy(xcache.at[pl.ds(0, CH)], rows[b], sem).wait()

    def multiply(j, b):
        def grp(g, icarry):
            wv = lg_v[pl.ds(j * CH + g * L, L)]
            for k in range(L):
                e = g * L + k
                w16 = wv.at[lane_idx[k]].get(mode='promise_in_bounds')
                for q in range(HALF // L):
                    sl = pl.ds(q * L, L)
                    rows[b][e, sl] = rows[b][e, sl] * w16
            return icarry

        lax.fori_loop(0, CH // L, grp, 0)

    def process(j, m, b):
        pltpu.make_async_copy(xcache.at[pl.ds(0, CH)], rows[b],
                              gsems[b]).wait()
        multiply(j, b)
        pltpu.async_copy(rows[b], acc.at[dstb.at[m, 0]], ssems[b],
                         add=True)

    for j in range(SD):
        issue_idx(j, j % SD)
    for b in range(A):
        wait_idx(b)
        issue_gather(b, b)

    T0 = (NCHUNK - 4) // UN

    def step(t, carry):
        for k in range(UN):
            j = t * UN + k
            b = k % NBUF
            bp = (b + A) % NBUF
            process(j, k % SD, b)
            if k == 0:
                @pl.when(t >= 1)
                def _drain0():
                    wait_rows_dma(bp, ssems[bp])
                    issue_idx(j + SD - 1, (k + SD - 1) % SD)
            else:
                wait_rows_dma(bp, ssems[bp])

                @pl.when(j + SD - 1 <= NCHUNK - 1)
                def _refill():
                    issue_idx(j + SD - 1, (k + SD - 1) % SD)

            wait_idx((k + A) % SD)
            issue_gather((k + A) % SD, bp)
        return carry

    lax.fori_loop(0, T0, step, 0)

    for j in range(NCHUNK - 4, NCHUNK):
        b = j % NBUF
        bp = (b + A) % NBUF
        process(j, j % SD, b)
        wait_rows_dma(bp, ssems[bp])
        if j + A <= NCHUNK - 1:
            wait_idx((j + A) % SD)
            issue_gather((j + A) % SD, bp)

    wait_rows_dma((NCHUNK - 1) % NBUF, ssems[(NCHUNK - 1) % NBUF])

    plsc.subcore_barrier()

    pltpu.sync_copy(acc.at[pl.ds(r0, RPT)],
                    out2.at[pl.ds(c * NP + r0, RPT)])


_sc_call = pl.kernel(
    _body,
    out_type=(
        jax.ShapeDtypeStruct((NC * NP, HALF), jnp.float32),
        jax.ShapeDtypeStruct((NC * NS * L,), jnp.float32),
    ),
    mesh=plsc.VectorSubcoreMesh(core_axis_name="c", subcore_axis_name="s"),
    compiler_params=pltpu.CompilerParams(
        needs_layout_passes=False, use_tc_tiling_on_sc=False),
    scratch_types=[
        pltpu.VMEM((EPT,), jnp.float32),
        pltpu.VMEM((SD, 1, CH), jnp.int32),
        pltpu.VMEM((SD, 1, CH), jnp.int32),
        pltpu.VMEM((CH,), jnp.float32),
        pltpu.VMEM((CH,), jnp.int32),
        pltpu.VMEM((CH,), jnp.int32),
        pltpu.VMEM((CH, HALF), jnp.float32),
        pltpu.VMEM((CH, HALF), jnp.float32),
        pltpu.VMEM((CH, HALF), jnp.float32),
        pltpu.VMEM((L,), jnp.float32),
        pltpu.VMEM_SHARED((NP, HALF), jnp.float32),
        pltpu.VMEM_SHARED((NP, HALF), jnp.float32),
        pltpu.SemaphoreType.DMA,
        pltpu.SemaphoreType.DMA,
        pltpu.SemaphoreType.DMA,
        pltpu.SemaphoreType.DMA,
        pltpu.SemaphoreType.DMA,
        pltpu.SemaphoreType.DMA,
        pltpu.SemaphoreType.DMA,
        pltpu.SemaphoreType.DMA,
        pltpu.SemaphoreType.DMA,
        pltpu.SemaphoreType.DMA,
        pltpu.SemaphoreType.DMA,
        pltpu.SemaphoreType.DMA,
    ],
)


def kernel(x, edge_index, gate_logits, feat_mask):
    x2 = x.reshape(NC * N, HALF)
    pad = EPAD - E
    src = jnp.concatenate([edge_index[0], jnp.zeros((pad,), jnp.int32)])
    dst = jnp.concatenate([edge_index[1], jnp.zeros((pad,), jnp.int32)])
    lg = jnp.concatenate(
        [gate_logits, jnp.full((pad,), -1e30, jnp.float32)])
    src2d = src.reshape(EPAD // CH, CH)
    dst2d = dst.reshape(EPAD // CH, CH)

    out2, pen = _sc_call(x2, src2d, dst2d, lg, feat_mask)
    out = jnp.concatenate([out2[:N], out2[NP:NP + N]], axis=1)
    penalty = jnp.sum(pen) / (NC * E)
    return out, penalty


# speedup vs baseline: 2.1382x; 1.0267x over previous
import math

import jax
import jax.numpy as jnp
from jax import lax
from jax.experimental import pallas as pl
from jax.experimental.pallas import tpu as pltpu
from jax.experimental.pallas import tpu_sc as plsc

N, E, D = 10000, 320000, 128
BETA = 1.0 / 3.0
GAMMA = -0.2
ZETA = 1.2
LOC_BIAS = 2.0
PEN_SHIFT = LOC_BIAS - BETA * math.log(-GAMMA / ZETA)

NC, NS, L = 2, 16, 16
CH = 128
NCHUNK = 160
EPT = NCHUNK * CH
EPAD = NS * EPT
HALF = D // 2
NP = 10240
RPT = NP // NS
NBUF = 3
A = 2
SD = 6
XB = RPT // CH
UN = 6


def _sigmoid(v):
    return 1.0 / (1.0 + jnp.exp(-v))


def _body(x2, srcp, dstp, lgp, fm, out2, pen,
          lg_v, srcb, dstb, fmb, xidx, fidx, rows0, rows1, rows2, pen_v,
          acc, xcache,
          gs0, gs1, gs2, ss0, ss1, ss2, is0, is1, is2, is3, is4, is5):
    c = lax.axis_index("c")
    s = lax.axis_index("s")
    ebase = s * EPT
    rows = (rows0, rows1, rows2)
    gsems = (gs0, gs1, gs2)
    ssems = (ss0, ss1, ss2)
    isems = (is0, is1, is2, is3, is4, is5)

    pltpu.async_copy(lgp.at[pl.ds(ebase, EPT)], lg_v, is0)

    lane_idx = [jnp.full((L,), k, jnp.int32) for k in range(L)]

    def drain_rows(buf, sem):
        pltpu.make_async_copy(xcache.at[pl.ds(0, CH)], buf, sem).wait()

    zero16 = jnp.zeros((L,), jnp.float32)

    def zrow(i, carry):
        for q in range(HALF // L):
            rows0[i, pl.ds(q * L, L)] = zero16
        return carry

    lax.fori_loop(0, CH, zrow, 0)
    r0 = s * RPT
    for k in range(RPT // CH):
        pltpu.async_copy(rows0, acc.at[pl.ds(r0 + k * CH, CH)], ss0)

    hb = (rows1, rows2)

    def issue_blk(blk):
        bb = blk % 2
        u0 = s * RPT + blk * CH

        def bidx(q, carry):
            io = lax.iota(jnp.int32, L)
            u = u0 + q * L + io
            xidx[bb, pl.ds(q * L, L)] = jnp.minimum(u * 2 + c, 2 * N - 1)
            fidx[bb, pl.ds(q * L, L)] = jnp.minimum(u, N - 1)
            return carry

        lax.fori_loop(0, CH // L, bidx, 0, unroll=True)
        pltpu.async_copy(x2.at[xidx.at[bb]], hb[bb], gsems[bb])
        pltpu.async_copy(fm.at[fidx.at[bb]], fmb.at[bb], gsems[bb])

    def finish_blk(blk):
        bb = blk % 2
        u0 = s * RPT + blk * CH
        pltpu.make_async_copy(x2.at[pl.ds(0, CH)], hb[bb],
                              gsems[bb]).wait()
        pltpu.make_async_copy(fm.at[pl.ds(0, CH)], fmb.at[bb],
                              gsems[bb]).wait()

        def sfm(q, carry):
            sl = pl.ds(q * L, L)
            fmb[bb, sl] = _sigmoid(fmb[bb, sl])
            return carry

        lax.fori_loop(0, CH // L, sfm, 0)

        def scale(g, carry):
            wv = fmb[bb, pl.ds(g * L, L)]
            for k in range(L):
                e = g * L + k
                w16 = wv.at[lane_idx[k]].get(mode='promise_in_bounds')
                for q in range(HALF // L):
                    sl = pl.ds(q * L, L)
                    hb[bb][e, sl] = hb[bb][e, sl] * w16
            return carry

        lax.fori_loop(0, CH // L, scale, 0)
        pltpu.async_copy(hb[bb], xcache.at[pl.ds(u0, CH)], ssems[1 + bb])

    issue_blk(0)
    issue_blk(1)
    finish_blk(0)
    for blk in range(2, XB):
        bb = blk % 2
        drain_rows(hb[bb], ssems[1 + bb])
        issue_blk(blk)
        finish_blk(blk - 1)
    finish_blk(XB - 1)
    drain_rows(hb[(XB - 2) % 2], ssems[1 + (XB - 2) % 2])
    drain_rows(hb[(XB - 1) % 2], ssems[1 + (XB - 1) % 2])

    pltpu.make_async_copy(lgp.at[pl.ds(0, EPT)], lg_v, is0).wait()
    PEN_K = math.exp(LOC_BIAS - PEN_SHIFT)

    def edge16(i, pacc):
        sl = pl.ds(i * L, L)
        lgv = lg_v[sl]
        a = jnp.exp(-(lgv + LOC_BIAS))
        g = 1.0 / (1.0 + a)
        lg_v[sl] = jnp.clip(g * (ZETA - GAMMA) + GAMMA, 0.0, 1.0)
        return pacc + 1.0 / (1.0 + a * PEN_K)

    pen16 = lax.fori_loop(0, EPT // L, edge16, jnp.zeros((L,), jnp.float32))
    pen_v[...] = pen16
    pltpu.sync_copy(pen_v, pen.at[pl.ds((c * NS + s) * L, L)])

    for k in range(RPT // CH):
        drain_rows(rows0, ss0)

    plsc.subcore_barrier()

    def issue_idx(j, m):
        pltpu.async_copy(srcp.at[pl.ds(s * NCHUNK + j, 1)], srcb.at[m],
                         isems[m])
        pltpu.async_copy(dstp.at[pl.ds(s * NCHUNK + j, 1)], dstb.at[m],
                         isems[m])

    def wait_idx(m):
        pltpu.make_async_copy(srcp.at[pl.ds(0, 1)], srcb.at[m],
                              isems[m]).wait()
        pltpu.make_async_copy(dstp.at[pl.ds(0, 1)], dstb.at[m],
                              isems[m]).wait()

    def issue_gather(m, b):
        pltpu.async_copy(xcache.at[srcb.at[m, 0]], rows[b], gsems[b])

    def wait_rows_dma(b, sem):
        pltpu.make_async_copy(xcache.at[pl.ds(0, CH)], rows[b], sem).wait()

    def multiply(j, b):
        def grp(g, icarry):
            wv = lg_v[pl.ds(j * CH + g * L, L)]
            for k in range(L):
                e = g * L + k
                w16 = wv.at[lane_idx[k]].get(mode='promise_in_bounds')
                for q in range(HALF // L):
                    sl = pl.ds(q * L, L)
                    rows[b][e, sl] = rows[b][e, sl] * w16
            return icarry

        lax.fori_loop(0, CH // L, grp, 0)

    def process(j, m, b):
        pltpu.make_async_copy(xcache.at[pl.ds(0, CH)], rows[b],
                              gsems[b]).wait()
        multiply(j, b)
        pltpu.async_copy(rows[b], acc.at[dstb.at[m, 0]], ssems[b],
                         add=True)

    for j in range(SD):
        issue_idx(j, j % SD)
    for b in range(A):
        wait_idx(b)
        issue_gather(b, b)

    T0 = (NCHUNK - 4) // UN

    def step(t, carry):
        for k in range(UN):
            j = t * UN + k
            b = k % NBUF
            bp = (b + A) % NBUF
            process(j, k % SD, b)
            if k == 0:
                @pl.when(t >= 1)
                def _drain0():
                    wait_rows_dma(bp, ssems[bp])
                    issue_idx(j + SD - 1, (k + SD - 1) % SD)
            else:
                wait_rows_dma(bp, ssems[bp])

                @pl.when(j + SD - 1 <= NCHUNK - 1)
                def _refill():
                    issue_idx(j + SD - 1, (k + SD - 1) % SD)

            wait_idx((k + A) % SD)
            issue_gather((k + A) % SD, bp)
        return carry

    lax.fori_loop(0, T0, step, 0)

    for j in range(NCHUNK - 4, NCHUNK):
        b = j % NBUF
        bp = (b + A) % NBUF
        process(j, j % SD, b)
        wait_rows_dma(bp, ssems[bp])
        if j + A <= NCHUNK - 1:
            wait_idx((j + A) % SD)
            issue_gather((j + A) % SD, bp)

    wait_rows_dma((NCHUNK - 1) % NBUF, ssems[(NCHUNK - 1) % NBUF])

    plsc.subcore_barrier()

    pltpu.sync_copy(acc.at[pl.ds(r0, RPT)],
                    out2.at[pl.ds(c * NP + r0, RPT)])


_sc_call = pl.kernel(
    _body,
    out_type=(
        jax.ShapeDtypeStruct((NC * NP, HALF), jnp.float32),
        jax.ShapeDtypeStruct((NC * NS * L,), jnp.float32),
    ),
    mesh=plsc.VectorSubcoreMesh(core_axis_name="c", subcore_axis_name="s"),
    compiler_params=pltpu.CompilerParams(
        needs_layout_passes=False, use_tc_tiling_on_sc=False),
    scratch_types=[
        pltpu.VMEM((EPT,), jnp.float32),
        pltpu.VMEM((SD, 1, CH), jnp.int32),
        pltpu.VMEM((SD, 1, CH), jnp.int32),
        pltpu.VMEM((2, CH), jnp.float32),
        pltpu.VMEM((2, CH), jnp.int32),
        pltpu.VMEM((2, CH), jnp.int32),
        pltpu.VMEM((CH, HALF), jnp.float32),
        pltpu.VMEM((CH, HALF), jnp.float32),
        pltpu.VMEM((CH, HALF), jnp.float32),
        pltpu.VMEM((L,), jnp.float32),
        pltpu.VMEM_SHARED((NP, HALF), jnp.float32),
        pltpu.VMEM_SHARED((NP, HALF), jnp.float32),
        pltpu.SemaphoreType.DMA,
        pltpu.SemaphoreType.DMA,
        pltpu.SemaphoreType.DMA,
        pltpu.SemaphoreType.DMA,
        pltpu.SemaphoreType.DMA,
        pltpu.SemaphoreType.DMA,
        pltpu.SemaphoreType.DMA,
        pltpu.SemaphoreType.DMA,
        pltpu.SemaphoreType.DMA,
        pltpu.SemaphoreType.DMA,
        pltpu.SemaphoreType.DMA,
        pltpu.SemaphoreType.DMA,
    ],
)


def kernel(x, edge_index, gate_logits, feat_mask):
    x2 = x.reshape(NC * N, HALF)
    pad = EPAD - E
    src = jnp.concatenate([edge_index[0], jnp.zeros((pad,), jnp.int32)])
    dst = jnp.concatenate([edge_index[1], jnp.zeros((pad,), jnp.int32)])
    lg = jnp.concatenate(
        [gate_logits, jnp.full((pad,), -1e30, jnp.float32)])
    src2d = src.reshape(EPAD // CH, CH)
    dst2d = dst.reshape(EPAD // CH, CH)

    out2, pen = _sc_call(x2, src2d, dst2d, lg, feat_mask)
    out = jnp.concatenate([out2[:N], out2[NP:NP + N]], axis=1)
    penalty = jnp.sum(pen) / (NC * E)
    return out, penalty


# speedup vs baseline: 2.3054x; 1.0782x over previous
import math

import jax
import jax.numpy as jnp
from jax import lax
from jax.experimental import pallas as pl
from jax.experimental.pallas import tpu as pltpu
from jax.experimental.pallas import tpu_sc as plsc

N, E, D = 10000, 320000, 128
BETA = 1.0 / 3.0
GAMMA = -0.2
ZETA = 1.2
LOC_BIAS = 2.0
PEN_SHIFT = LOC_BIAS - BETA * math.log(-GAMMA / ZETA)

NC, NS, L = 2, 16, 16
CH = 128
NCHUNK = 160
EPT = NCHUNK * CH
EPAD = NS * EPT
HALF = D // 2
NP = 10240
RPT = NP // NS
NBUF = 3
A = 2
SD = 6
XB = RPT // CH
UN = 6


def _sigmoid(v):
    return 1.0 / (1.0 + jnp.exp(-v))


def _body(x2, srcp, dstp, lgp, fm, out2, pen,
          lg_v, srcb, dstb, fmb, xidx, fidx, rows0, rows1, rows2, pen_v,
          acc, xcache,
          gs0, gs1, gs2, ss0, ss1, ss2, is0, is1, is2, is3, is4, is5):
    c = lax.axis_index("c")
    s = lax.axis_index("s")
    ebase = s * EPT
    rows = (rows0, rows1, rows2)
    gsems = (gs0, gs1, gs2)
    ssems = (ss0, ss1, ss2)
    isems = (is0, is1, is2, is3, is4, is5)

    pltpu.async_copy(lgp.at[pl.ds(ebase, EPT)], lg_v, is0)

    lane_idx = [jnp.full((L,), k, jnp.int32) for k in range(L)]

    def drain_rows(buf, sem):
        pltpu.make_async_copy(xcache.at[pl.ds(0, CH)], buf, sem).wait()

    zero16 = jnp.zeros((L,), jnp.float32)

    def zrow(i, carry):
        for q in range(HALF // L):
            rows0[i, pl.ds(q * L, L)] = zero16
        return carry

    lax.fori_loop(0, CH, zrow, 0)
    r0 = s * RPT
    for k in range(RPT // CH):
        pltpu.async_copy(rows0, acc.at[pl.ds(r0 + k * CH, CH)], ss0)

    hb = (rows1, rows2)

    def issue_blk(blk):
        bb = blk % 2
        u0 = s * RPT + blk * CH

        def bidx(q, carry):
            io = lax.iota(jnp.int32, L)
            u = u0 + q * L + io
            xidx[bb, pl.ds(q * L, L)] = jnp.minimum(u * 2 + c, 2 * N - 1)
            fidx[bb, pl.ds(q * L, L)] = jnp.minimum(u, N - 1)
            return carry

        lax.fori_loop(0, CH // L, bidx, 0, unroll=True)
        pltpu.async_copy(x2.at[xidx.at[bb]], hb[bb], gsems[bb])
        pltpu.async_copy(fm.at[fidx.at[bb]], fmb.at[bb], gsems[bb])

    def finish_blk(blk):
        bb = blk % 2
        u0 = s * RPT + blk * CH
        pltpu.make_async_copy(x2.at[pl.ds(0, CH)], hb[bb],
                              gsems[bb]).wait()
        pltpu.make_async_copy(fm.at[pl.ds(0, CH)], fmb.at[bb],
                              gsems[bb]).wait()

        def sfm(q, carry):
            sl = pl.ds(q * L, L)
            fmb[bb, sl] = _sigmoid(fmb[bb, sl])
            return carry

        lax.fori_loop(0, CH // L, sfm, 0)

        def scale(g, carry):
            wv = fmb[bb, pl.ds(g * L, L)]
            for k in range(L):
                e = g * L + k
                w16 = wv.at[lane_idx[k]].get(mode='promise_in_bounds')
                for q in range(HALF // L):
                    sl = pl.ds(q * L, L)
                    hb[bb][e, sl] = hb[bb][e, sl] * w16
            return carry

        lax.fori_loop(0, CH // L, scale, 0)
        pltpu.async_copy(hb[bb], xcache.at[pl.ds(u0, CH)], ssems[1 + bb])

    issue_blk(0)
    issue_blk(1)
    finish_blk(0)
    for blk in range(2, XB):
        bb = blk % 2
        drain_rows(hb[bb], ssems[1 + bb])
        issue_blk(blk)
        finish_blk(blk - 1)
    finish_blk(XB - 1)
    drain_rows(hb[(XB - 2) % 2], ssems[1 + (XB - 2) % 2])
    drain_rows(hb[(XB - 1) % 2], ssems[1 + (XB - 1) % 2])

    pltpu.make_async_copy(lgp.at[pl.ds(0, EPT)], lg_v, is0).wait()
    PEN_K = math.exp(LOC_BIAS - PEN_SHIFT)

    def edge16(i, pacc):
        sl = pl.ds(i * L, L)
        lgv = lg_v[sl]
        a = jnp.exp(-(lgv + LOC_BIAS))
        g = 1.0 / (1.0 + a)
        lg_v[sl] = jnp.clip(g * (ZETA - GAMMA) + GAMMA, 0.0, 1.0)
        return pacc + 1.0 / (1.0 + a * PEN_K)

    pen16 = lax.fori_loop(0, EPT // L, edge16, jnp.zeros((L,), jnp.float32),
                          unroll=4)
    pen_v[...] = pen16
    pltpu.sync_copy(pen_v, pen.at[pl.ds((c * NS + s) * L, L)])

    for k in range(RPT // CH):
        drain_rows(rows0, ss0)

    plsc.subcore_barrier()

    def issue_idx(j, m):
        pltpu.async_copy(srcp.at[pl.ds(s * NCHUNK + j, 1)], srcb.at[m],
                         isems[m])
        pltpu.async_copy(dstp.at[pl.ds(s * NCHUNK + j, 1)], dstb.at[m],
                         isems[m])

    def wait_idx(m):
        pltpu.make_async_copy(srcp.at[pl.ds(0, 1)], srcb.at[m],
                              isems[m]).wait()
        pltpu.make_async_copy(dstp.at[pl.ds(0, 1)], dstb.at[m],
                              isems[m]).wait()

    def issue_gather(m, b):
        pltpu.async_copy(xcache.at[srcb.at[m, 0]], rows[b], gsems[b])

    def wait_rows_dma(b, sem):
        pltpu.make_async_copy(xcache.at[pl.ds(0, CH)], rows[b], sem).wait()

    def multiply(j, b):
        def grp(g, icarry):
            wv = lg_v[pl.ds(j * CH + g * L, L)]
            for k in range(L):
                e = g * L + k
                w16 = wv.at[lane_idx[k]].get(mode='promise_in_bounds')
                for q in range(HALF // L):
                    sl = pl.ds(q * L, L)
                    rows[b][e, sl] = rows[b][e, sl] * w16
            return icarry

        lax.fori_loop(0, CH // L, grp, 0)

    def process(j, m, b):
        pltpu.make_async_copy(xcache.at[pl.ds(0, CH)], rows[b],
                              gsems[b]).wait()
        multiply(j, b)
        pltpu.async_copy(rows[b], acc.at[dstb.at[m, 0]], ssems[b],
                         add=True)

    for j in range(SD):
        issue_idx(j, j % SD)
    for b in range(A):
        wait_idx(b)
        issue_gather(b, b)

    T0 = (NCHUNK - 4) // UN

    def step(t, carry):
        for k in range(UN):
            j = t * UN + k
            b = k % NBUF
            bp = (b + A) % NBUF
            process(j, k % SD, b)
            if k == 0:
                @pl.when(t >= 1)
                def _drain0():
                    wait_rows_dma(bp, ssems[bp])
                    issue_idx(j + SD - 1, (k + SD - 1) % SD)
            else:
                wait_rows_dma(bp, ssems[bp])

                @pl.when(j + SD - 1 <= NCHUNK - 1)
                def _refill():
                    issue_idx(j + SD - 1, (k + SD - 1) % SD)

            wait_idx((k + A) % SD)
            issue_gather((k + A) % SD, bp)
        return carry

    lax.fori_loop(0, T0, step, 0)

    for j in range(NCHUNK - 4, NCHUNK):
        b = j % NBUF
        bp = (b + A) % NBUF
        process(j, j % SD, b)
        wait_rows_dma(bp, ssems[bp])
        if j + A <= NCHUNK - 1:
            wait_idx((j + A) % SD)
            issue_gather((j + A) % SD, bp)

    wait_rows_dma((NCHUNK - 1) % NBUF, ssems[(NCHUNK - 1) % NBUF])

    plsc.subcore_barrier()

    pltpu.sync_copy(acc.at[pl.ds(r0, RPT)],
                    out2.at[pl.ds(c * NP + r0, RPT)])


_sc_call = pl.kernel(
    _body,
    out_type=(
        jax.ShapeDtypeStruct((NC * NP, HALF), jnp.float32),
        jax.ShapeDtypeStruct((NC * NS * L,), jnp.float32),
    ),
    mesh=plsc.VectorSubcoreMesh(core_axis_name="c", subcore_axis_name="s"),
    compiler_params=pltpu.CompilerParams(
        needs_layout_passes=False, use_tc_tiling_on_sc=False),
    scratch_types=[
        pltpu.VMEM((EPT,), jnp.float32),
        pltpu.VMEM((SD, 1, CH), jnp.int32),
        pltpu.VMEM((SD, 1, CH), jnp.int32),
        pltpu.VMEM((2, CH), jnp.float32),
        pltpu.VMEM((2, CH), jnp.int32),
        pltpu.VMEM((2, CH), jnp.int32),
        pltpu.VMEM((CH, HALF), jnp.float32),
        pltpu.VMEM((CH, HALF), jnp.float32),
        pltpu.VMEM((CH, HALF), jnp.float32),
        pltpu.VMEM((L,), jnp.float32),
        pltpu.VMEM_SHARED((NP, HALF), jnp.float32),
        pltpu.VMEM_SHARED((NP, HALF), jnp.float32),
        pltpu.SemaphoreType.DMA,
        pltpu.SemaphoreType.DMA,
        pltpu.SemaphoreType.DMA,
        pltpu.SemaphoreType.DMA,
        pltpu.SemaphoreType.DMA,
        pltpu.SemaphoreType.DMA,
        pltpu.SemaphoreType.DMA,
        pltpu.SemaphoreType.DMA,
        pltpu.SemaphoreType.DMA,
        pltpu.SemaphoreType.DMA,
        pltpu.SemaphoreType.DMA,
        pltpu.SemaphoreType.DMA,
    ],
)


def kernel(x, edge_index, gate_logits, feat_mask):
    x2 = x.reshape(NC * N, HALF)
    pad = EPAD - E
    src = jnp.concatenate([edge_index[0], jnp.zeros((pad,), jnp.int32)])
    dst = jnp.concatenate([edge_index[1], jnp.zeros((pad,), jnp.int32)])
    lg = jnp.concatenate(
        [gate_logits, jnp.full((pad,), -1e30, jnp.float32)])
    src2d = src.reshape(EPAD // CH, CH)
    dst2d = dst.reshape(EPAD // CH, CH)

    out2, pen = _sc_call(x2, src2d, dst2d, lg, feat_mask)
    out = jnp.concatenate([out2[:N], out2[NP:NP + N]], axis=1)
    penalty = jnp.sum(pen) / (NC * E)
    return out, penalty


# speedup vs baseline: 2.6200x; 1.1365x over previous
import math

import jax
import jax.numpy as jnp
from jax import lax
from jax.experimental import pallas as pl
from jax.experimental.pallas import tpu as pltpu
from jax.experimental.pallas import tpu_sc as plsc

N, E, D = 10000, 320000, 128
BETA = 1.0 / 3.0
GAMMA = -0.2
ZETA = 1.2
LOC_BIAS = 2.0
PEN_SHIFT = LOC_BIAS - BETA * math.log(-GAMMA / ZETA)

NC, NS, L = 2, 16, 16
CH = 128
NCHUNK = 160
EPT = NCHUNK * CH
EPAD = NS * EPT
HALF = D // 2
NP = 10240
RPT = NP // NS
NBUF = 3
A = 2
SD = 6
XB = RPT // CH
UN = 6


def _sigmoid(v):
    return 1.0 / (1.0 + jnp.exp(-v))


def _body(x2, srcp, dstp, lgp, fm, out2, pen,
          lg_v, srcb, dstb, fmb, xidx, fidx, rows0, rows1, rows2, pen_v,
          acc, xcache,
          gs0, gs1, gs2, ss0, ss1, ss2, is0, is1, is2, is3, is4, is5):
    c = lax.axis_index("c")
    s = lax.axis_index("s")
    ebase = s * EPT
    rows = (rows0, rows1, rows2)
    gsems = (gs0, gs1, gs2)
    ssems = (ss0, ss1, ss2)
    isems = (is0, is1, is2, is3, is4, is5)

    pltpu.async_copy(lgp.at[pl.ds(ebase, EPT)], lg_v, is0)

    lane_idx = [jnp.full((L,), k, jnp.int32) for k in range(L)]

    def drain_rows(buf, sem):
        pltpu.make_async_copy(xcache.at[pl.ds(0, CH)], buf, sem).wait()

    zero16 = jnp.zeros((L,), jnp.float32)

    def zrow(i, carry):
        for q in range(HALF // L):
            rows0[i, pl.ds(q * L, L)] = zero16
        return carry

    lax.fori_loop(0, CH, zrow, 0)
    r0 = s * RPT
    for k in range(RPT // CH):
        pltpu.async_copy(rows0, acc.at[pl.ds(r0 + k * CH, CH)], ss0)

    hb = (rows1, rows2)

    def issue_blk(blk):
        bb = blk % 2
        u0 = s * RPT + blk * CH

        def bidx(q, carry):
            io = lax.iota(jnp.int32, L)
            u = u0 + q * L + io
            xidx[bb, pl.ds(q * L, L)] = jnp.minimum(u * 2 + c, 2 * N - 1)
            fidx[bb, pl.ds(q * L, L)] = jnp.minimum(u, N - 1)
            return carry

        lax.fori_loop(0, CH // L, bidx, 0, unroll=True)
        pltpu.async_copy(x2.at[xidx.at[bb]], hb[bb], gsems[bb])
        pltpu.async_copy(fm.at[fidx.at[bb]], fmb.at[bb], gsems[bb])

    def finish_blk(blk):
        bb = blk % 2
        u0 = s * RPT + blk * CH
        pltpu.make_async_copy(x2.at[pl.ds(0, CH)], hb[bb],
                              gsems[bb]).wait()
        pltpu.make_async_copy(fm.at[pl.ds(0, CH)], fmb.at[bb],
                              gsems[bb]).wait()

        def sfm(q, carry):
            sl = pl.ds(q * L, L)
            fmb[bb, sl] = _sigmoid(fmb[bb, sl])
            return carry

        lax.fori_loop(0, CH // L, sfm, 0)

        def scale(g, carry):
            wv = fmb[bb, pl.ds(g * L, L)]
            for k in range(L):
                e = g * L + k
                w16 = wv.at[lane_idx[k]].get(mode='promise_in_bounds')
                for q in range(HALF // L):
                    sl = pl.ds(q * L, L)
                    hb[bb][e, sl] = hb[bb][e, sl] * w16
            return carry

        lax.fori_loop(0, CH // L, scale, 0)
        pltpu.async_copy(hb[bb], xcache.at[pl.ds(u0, CH)], ssems[1 + bb])

    issue_blk(0)
    issue_blk(1)
    finish_blk(0)
    for blk in range(2, XB):
        bb = blk % 2
        drain_rows(hb[bb], ssems[1 + bb])
        issue_blk(blk)
        finish_blk(blk - 1)
    finish_blk(XB - 1)
    drain_rows(hb[(XB - 2) % 2], ssems[1 + (XB - 2) % 2])
    drain_rows(hb[(XB - 1) % 2], ssems[1 + (XB - 1) % 2])

    pltpu.make_async_copy(lgp.at[pl.ds(0, EPT)], lg_v, is0).wait()
    PEN_K = math.exp(LOC_BIAS - PEN_SHIFT)

    def edge16(i, pacc):
        sl = pl.ds(i * L, L)
        lgv = lg_v[sl]
        a = jnp.exp(-(lgv + LOC_BIAS))
        g = 1.0 / (1.0 + a)
        lg_v[sl] = jnp.clip(g * (ZETA - GAMMA) + GAMMA, 0.0, 1.0)
        return pacc + 1.0 / (1.0 + a * PEN_K)

    pen16 = lax.fori_loop(0, EPT // L, edge16, jnp.zeros((L,), jnp.float32),
                          unroll=8)
    pen_v[...] = pen16
    pltpu.sync_copy(pen_v, pen.at[pl.ds((c * NS + s) * L, L)])

    for k in range(RPT // CH):
        drain_rows(rows0, ss0)

    plsc.subcore_barrier()

    def issue_idx(j, m):
        pltpu.async_copy(srcp.at[pl.ds(s * NCHUNK + j, 1)], srcb.at[m],
                         isems[m])
        pltpu.async_copy(dstp.at[pl.ds(s * NCHUNK + j, 1)], dstb.at[m],
                         isems[m])

    def wait_idx(m):
        pltpu.make_async_copy(srcp.at[pl.ds(0, 1)], srcb.at[m],
                              isems[m]).wait()
        pltpu.make_async_copy(dstp.at[pl.ds(0, 1)], dstb.at[m],
                              isems[m]).wait()

    def issue_gather(m, b):
        pltpu.async_copy(xcache.at[srcb.at[m, 0]], rows[b], gsems[b])

    def wait_rows_dma(b, sem):
        pltpu.make_async_copy(xcache.at[pl.ds(0, CH)], rows[b], sem).wait()

    def multiply(j, b):
        def grp(g, icarry):
            wv = lg_v[pl.ds(j * CH + g * L, L)]
            for k in range(L):
                e = g * L + k
                w16 = wv.at[lane_idx[k]].get(mode='promise_in_bounds')
                for q in range(HALF // L):
                    sl = pl.ds(q * L, L)
                    rows[b][e, sl] = rows[b][e, sl] * w16
            return icarry

        lax.fori_loop(0, CH // L, grp, 0, unroll=2)

    def process(j, m, b):
        pltpu.make_async_copy(xcache.at[pl.ds(0, CH)], rows[b],
                              gsems[b]).wait()
        multiply(j, b)
        pltpu.async_copy(rows[b], acc.at[dstb.at[m, 0]], ssems[b],
                         add=True)

    for j in range(SD):
        issue_idx(j, j % SD)
    for b in range(A):
        wait_idx(b)
        issue_gather(b, b)

    T0 = (NCHUNK - 4) // UN

    def step(t, carry):
        for k in range(UN):
            j = t * UN + k
            b = k % NBUF
            bp = (b + A) % NBUF
            process(j, k % SD, b)
            if k == 0:
                @pl.when(t >= 1)
                def _drain0():
                    wait_rows_dma(bp, ssems[bp])
                    issue_idx(j + SD - 1, (k + SD - 1) % SD)
            else:
                wait_rows_dma(bp, ssems[bp])

                @pl.when(j + SD - 1 <= NCHUNK - 1)
                def _refill():
                    issue_idx(j + SD - 1, (k + SD - 1) % SD)

            wait_idx((k + A) % SD)
            issue_gather((k + A) % SD, bp)
        return carry

    lax.fori_loop(0, T0, step, 0)

    for j in range(NCHUNK - 4, NCHUNK):
        b = j % NBUF
        bp = (b + A) % NBUF
        process(j, j % SD, b)
        wait_rows_dma(bp, ssems[bp])
        if j + A <= NCHUNK - 1:
            wait_idx((j + A) % SD)
            issue_gather((j + A) % SD, bp)

    wait_rows_dma((NCHUNK - 1) % NBUF, ssems[(NCHUNK - 1) % NBUF])

    plsc.subcore_barrier()

    pltpu.sync_copy(acc.at[pl.ds(r0, RPT)],
                    out2.at[pl.ds(c * NP + r0, RPT)])


_sc_call = pl.kernel(
    _body,
    out_type=(
        jax.ShapeDtypeStruct((NC * NP, HALF), jnp.float32),
        jax.ShapeDtypeStruct((NC * NS * L,), jnp.float32),
    ),
    mesh=plsc.VectorSubcoreMesh(core_axis_name="c", subcore_axis_name="s"),
    compiler_params=pltpu.CompilerParams(
        needs_layout_passes=False, use_tc_tiling_on_sc=False),
    scratch_types=[
        pltpu.VMEM((EPT,), jnp.float32),
        pltpu.VMEM((SD, 1, CH), jnp.int32),
        pltpu.VMEM((SD, 1, CH), jnp.int32),
        pltpu.VMEM((2, CH), jnp.float32),
        pltpu.VMEM((2, CH), jnp.int32),
        pltpu.VMEM((2, CH), jnp.int32),
        pltpu.VMEM((CH, HALF), jnp.float32),
        pltpu.VMEM((CH, HALF), jnp.float32),
        pltpu.VMEM((CH, HALF), jnp.float32),
        pltpu.VMEM((L,), jnp.float32),
        pltpu.VMEM_SHARED((NP, HALF), jnp.float32),
        pltpu.VMEM_SHARED((NP, HALF), jnp.float32),
        pltpu.SemaphoreType.DMA,
        pltpu.SemaphoreType.DMA,
        pltpu.SemaphoreType.DMA,
        pltpu.SemaphoreType.DMA,
        pltpu.SemaphoreType.DMA,
        pltpu.SemaphoreType.DMA,
        pltpu.SemaphoreType.DMA,
        pltpu.SemaphoreType.DMA,
        pltpu.SemaphoreType.DMA,
        pltpu.SemaphoreType.DMA,
        pltpu.SemaphoreType.DMA,
        pltpu.SemaphoreType.DMA,
    ],
)


def kernel(x, edge_index, gate_logits, feat_mask):
    x2 = x.reshape(NC * N, HALF)
    pad = EPAD - E
    src = jnp.concatenate([edge_index[0], jnp.zeros((pad,), jnp.int32)])
    dst = jnp.concatenate([edge_index[1], jnp.zeros((pad,), jnp.int32)])
    lg = jnp.concatenate(
        [gate_logits, jnp.full((pad,), -1e30, jnp.float32)])
    src2d = src.reshape(EPAD // CH, CH)
    dst2d = dst.reshape(EPAD // CH, CH)

    out2, pen = _sc_call(x2, src2d, dst2d, lg, feat_mask)
    out = jnp.concatenate([out2[:N], out2[NP:NP + N]], axis=1)
    penalty = jnp.sum(pen) / (NC * E)
    return out, penalty
